# baseline (speedup 1.0000x reference)
import functools

import numpy as np
import jax
import jax.numpy as jnp
from jax import lax
from jax.experimental import pallas as pl
from jax.experimental.pallas import tpu as pltpu
from jax.experimental.pallas import tpu_sc as plsc

N = 50
NPAIR = N * (N - 1) // 2
PP = 1280
B = 4096
NW = 32
ROWS_W = B // NW
L = 16

_LN2 = float(np.log(2.0))


def _pair_indices():
    ii = np.zeros(PP, np.int32)
    jj = np.zeros(PP, np.int32)
    p = 0
    for i in range(N):
        for j in range(i + 1, N):
            ii[p] = i
            jj[p] = j
            p += 1
    return ii, jj


_I_NP, _J_NP = _pair_indices()

_ZG = np.linspace(0.0, 1.0, 4097)
_PC = np.polyfit(_ZG, np.log2(1.0 + _ZG), 10).astype(np.float32)

_MANT = np.int32(0x007FFFFF)
_ONEBITS = np.int32(0x3F800000)

_mesh = plsc.VectorSubcoreMesh(core_axis_name="c", subcore_axis_name="s")


@functools.partial(
    pl.kernel,
    mesh=_mesh,
    out_type=jax.ShapeDtypeStruct((NW, L), jnp.float32),
    scratch_types=[
        pltpu.VMEM((ROWS_W * N + L,), jnp.float32),
        pltpu.VMEM((ROWS_W * N + L,), jnp.float32),
        pltpu.VMEM((L,), jnp.float32),
    ],
)
def _sc_kernel(s_hbm, t_hbm, out_hbm, s_v, t_v, res_v):
    wid = lax.axis_index("s") * 2 + lax.axis_index("c")
    base = wid * ROWS_W * N
    pltpu.sync_copy(s_hbm.at[pl.ds(base, ROWS_W * N)],
                    s_v.at[pl.ds(0, ROWS_W * N)])
    pltpu.sync_copy(t_hbm.at[pl.ds(base, ROWS_W * N)],
                    t_v.at[pl.ds(0, ROWS_W * N)])

    lanes = lax.iota(jnp.int32, L)

    def row_body(r, carry):
        prod, eacc, linacc = carry
        rb = r * N

        def make_chunk(c):
            def chunk_body(k, carry2):
                prod, eacc, linacc = carry2
                o1 = rb + L * c
                o2 = o1 + k
                si = s_v[pl.ds(o1, L)]
                sj = s_v[pl.ds(o2, L)]
                ti = t_v[pl.ds(o1, L)]
                tj = t_v[pl.ds(o2, L)]
                mask = lanes < (N - k - L * c)
                d0 = si - sj
                x = d0 * jnp.sign(ti - tj)
                q = 1.0 + jnp.exp(-jnp.abs(d0))
                prod = prod * jnp.where(mask, q, 1.0)
                linacc = linacc + jnp.where(mask, jnp.maximum(-x, 0.0), 0.0)
                return prod, eacc, linacc
            return chunk_body

        prod, eacc, linacc = lax.fori_loop(1, N, make_chunk(0),
                                           (prod, eacc, linacc))
        prod, eacc, linacc = lax.fori_loop(1, N - L, make_chunk(1),
                                           (prod, eacc, linacc))
        prod, eacc, linacc = lax.fori_loop(1, N - 2 * L, make_chunk(2),
                                           (prod, eacc, linacc))
        prod, eacc, linacc = make_chunk(3)(1, (prod, eacc, linacc))
        bits = lax.bitcast_convert_type(prod, jnp.int32)
        e = lax.shift_right_logical(bits, 23) - 127
        eacc = eacc + e.astype(jnp.float32)
        prod = lax.bitcast_convert_type((bits & _MANT) | _ONEBITS, jnp.float32)
        return prod, eacc, linacc

    prod, eacc, linacc = lax.fori_loop(
        0, ROWS_W, row_body,
        (jnp.ones((L,), jnp.float32), jnp.zeros((L,), jnp.float32),
         jnp.zeros((L,), jnp.float32)))

    z = prod - 1.0
    poly = jnp.zeros((L,), jnp.float32) + np.float32(_PC[0])
    for c in _PC[1:]:
        poly = poly * z + np.float32(c)
    total = (eacc + poly) * np.float32(_LN2) + linacc
    res_v[...] = total
    pltpu.sync_copy(res_v, out_hbm.at[wid])


@functools.partial(jax.jit, static_argnames=())
def kernel(student_scores, teacher_scores):
    partial = _sc_kernel(student_scores.reshape(-1), teacher_scores.reshape(-1))
    return jnp.sum(partial) / np.float32(NPAIR * B)


# speedup vs baseline: 6.4864x; 6.4864x over previous
import functools

import numpy as np
import jax
import jax.numpy as jnp
from jax.experimental import pallas as pl
from jax.experimental.pallas import tpu as pltpu

N = 50
NPAIR = N * (N - 1) // 2
P = 1280
B = 4096
BS = 2048

_LOG2E = float(np.log2(np.e))
_LN2 = float(np.log(2.0))


def _pair_diff_matrix() -> np.ndarray:
    d = np.zeros((N, P), np.float32)
    p = 0
    for i in range(N):
        for j in range(i + 1, N):
            d[i, p] = 1.0
            d[j, p] = -1.0
            p += 1
    return d


_D_NP = _pair_diff_matrix()
_SIGNBIT = np.int32(-2147483648)


def _body(s_ref, t_ref, d_ref, out_ref, acc_sum):
    s = (s_ref[...] * np.float32(-_LOG2E)).astype(jnp.bfloat16)
    t = t_ref[...].astype(jnp.bfloat16)
    d1 = jnp.dot(s, d_ref[...], preferred_element_type=jnp.float32)
    dt = jnp.dot(t, d_ref[...], preferred_element_type=jnp.float32)
    d1b = jax.lax.bitcast_convert_type(d1, jnp.int32)
    dtb = jax.lax.bitcast_convert_type(dt, jnp.int32)
    m = jax.lax.bitcast_convert_type(d1b | _SIGNBIT, jnp.float32)
    q = 1.0 + jax.lax.exp2(m)
    sd1 = jax.lax.bitcast_convert_type(d1b ^ (dtb & _SIGNBIT), jnp.float32)
    lin = jnp.maximum(sd1, 0.0)
    psum = lin.reshape(BS // 8, 8, P).sum(axis=0)
    q3 = q.reshape(BS // 8, 8, P)
    for c in range(BS // 8 // 64):
        vs = [q3[c * 64 + i] for i in range(64)]
        while len(vs) > 1:
            vs = [a * b for a, b in zip(vs[::2], vs[1::2])]
        bits = jax.lax.bitcast_convert_type(vs[0], jnp.int32)
        e = (jax.lax.shift_right_logical(bits, 23) - 127).astype(jnp.float32)
        mant = jax.lax.bitcast_convert_type(
            (bits & np.int32(0x007FFFFF)) | np.int32(0x3F800000), jnp.float32)
        psum = psum + (e + jnp.log2(mant))

    @pl.when(pl.program_id(0) == 0)
    def _():
        acc_sum[...] = jnp.zeros((8, P), jnp.float32)

    acc_sum[...] += psum

    @pl.when(pl.program_id(0) == pl.num_programs(0) - 1)
    def _():
        col = jax.lax.broadcasted_iota(jnp.int32, (8, P), 1)
        tot = jnp.sum(jnp.where(col < NPAIR, acc_sum[...], 0.0))
        out_ref[0] = tot * np.float32(_LN2 / (NPAIR * B))


@functools.partial(jax.jit, static_argnames=())
def kernel(student_scores, teacher_scores):
    dmat = jnp.asarray(_D_NP, dtype=jnp.bfloat16)
    out = pl.pallas_call(
        _body,
        grid=(B // BS,),
        in_specs=[
            pl.BlockSpec((BS, N), lambda i: (i, 0)),
            pl.BlockSpec((BS, N), lambda i: (i, 0)),
            pl.BlockSpec((N, P), lambda i: (0, 0)),
        ],
        out_specs=pl.BlockSpec(memory_space=pltpu.SMEM),
        out_shape=jax.ShapeDtypeStruct((1,), jnp.float32),
        scratch_shapes=[
            pltpu.VMEM((8, P), jnp.float32),
        ],
    )(student_scores, teacher_scores, dmat)
    return out[0]
